# baseline (speedup 1.0000x reference)
import functools

import jax
import jax.numpy as jnp
from jax import lax
from jax.experimental import pallas as pl
from jax.experimental.pallas import tpu as pltpu
from jax.experimental.pallas import tpu_sc as plsc

N_NODES = 10000
N_EDGES = 320000
IN_FEAT = 128
HIDDEN = 128
NUM_CLASSES = 16
HALF = HIDDEN // 2

NC = 2
NS = 16
NW = NC * NS
EPT = N_EDGES // NW
K_CHUNK = 128
CPT = 80
EPT_PAD = CPT * K_CHUNK
E_PAD = NW * EPT_PAD
N_ACC = N_NODES + 16
ROWS_PT = 624
ROWS_REM = N_NODES - NS * ROWS_PT
REM_BASE = NS * ROWS_PT

_MESH = plsc.VectorSubcoreMesh(core_axis_name="c", subcore_axis_name="s")


@functools.partial(
    pl.kernel,
    out_type=jax.ShapeDtypeStruct((NW, N_NODES), jnp.float32),
    mesh=_MESH,
    scratch_types=[
        pltpu.VMEM((EPT,), jnp.int32),
        pltpu.VMEM((N_NODES,), jnp.float32),
    ],
    compiler_params=pltpu.CompilerParams(needs_layout_passes=False),
)
def _deg_kernel(dst_hbm, out_hbm, idx_v, deg_v):
    c = lax.axis_index("c")
    s = lax.axis_index("s")
    wid = c * NS + s
    base = wid * EPT
    pltpu.sync_copy(dst_hbm.at[pl.ds(base, EPT)], idx_v)

    zeros16 = jnp.zeros((16,), jnp.float32)

    def zero_body(i, carry):
        deg_v[pl.ds(i * 16, 16)] = zeros16
        return carry

    lax.fori_loop(0, N_NODES // 16, zero_body, 0)

    ones16 = jnp.ones((16,), jnp.float32)

    def add_body(i, carry):
        idx = idx_v[pl.ds(i * 16, 16)]
        plsc.addupdate_scatter(deg_v, [idx], ones16)
        return carry

    lax.fori_loop(0, EPT // 16, add_body, 0)
    pltpu.sync_copy(deg_v, out_hbm.at[wid])


def _make_edge_scatter(feat, npass):
    @functools.partial(
        pl.kernel,
        out_type=jax.ShapeDtypeStruct((npass, NC, N_NODES, feat), jnp.float32),
        mesh=_MESH,
        compiler_params=pltpu.CompilerParams(use_tc_tiling_on_sc=False),
        scratch_types=[
            pltpu.VMEM((CPT, K_CHUNK), jnp.int32),
            pltpu.VMEM((CPT, K_CHUNK), jnp.int32),
            pltpu.VMEM((2, K_CHUNK, feat), jnp.float32),
            pltpu.VMEM_SHARED((N_NODES, feat), jnp.float32),
            pltpu.VMEM_SHARED((N_ACC, feat), jnp.float32),
            pltpu.SemaphoreType.DMA,
            pltpu.SemaphoreType.DMA,
        ],
    )
    def edge_kernel(y_hbm, src_hbm, dst_hbm, zeros_hbm, out_hbm,
                    sidx, didx, rows, y_s, acc, sem0, sem1):
        c = lax.axis_index("c")
        s = lax.axis_index("s")
        wid = c * NS + s
        rbase = s * ROWS_PT

        pltpu.sync_copy(src_hbm.at[wid], sidx)
        pltpu.sync_copy(dst_hbm.at[wid], didx)

        def fire(g, buf, sem):
            pltpu.async_copy(y_s.at[sidx.at[g]], rows.at[buf], sem)

        def wait(g, buf, sem):
            pltpu.make_async_copy(y_s.at[sidx.at[g]], rows.at[buf], sem).wait()

        def scatter(g, buf):
            pltpu.sync_copy(rows.at[buf], acc.at[didx.at[g]], add=True)

        for p in range(npass):
            pltpu.sync_copy(y_hbm.at[p, pl.ds(rbase, ROWS_PT)],
                            y_s.at[pl.ds(rbase, ROWS_PT)])
            pltpu.sync_copy(zeros_hbm, acc.at[pl.ds(rbase, ROWS_PT)])

            @pl.when(s == NS - 1)
            def _():
                pltpu.sync_copy(y_hbm.at[p, pl.ds(REM_BASE, ROWS_REM)],
                                y_s.at[pl.ds(REM_BASE, ROWS_REM)])
                pltpu.sync_copy(zeros_hbm.at[pl.ds(0, ROWS_REM)],
                                acc.at[pl.ds(REM_BASE, ROWS_REM)])

            plsc.subcore_barrier()

            fire(0, 0, sem0)

            def body(q, carry):
                g0 = 2 * q
                g1 = g0 + 1
                fire(g1, 1, sem1)
                wait(g0, 0, sem0)
                scatter(g0, 0)

                @pl.when(g1 + 1 < CPT)
                def _():
                    fire(g1 + 1, 0, sem0)

                wait(g1, 1, sem1)
                scatter(g1, 1)
                return carry

            lax.fori_loop(0, CPT // 2, body, 0)
            plsc.subcore_barrier()
            pltpu.sync_copy(acc.at[pl.ds(rbase, ROWS_PT)],
                            out_hbm.at[p, c, pl.ds(rbase, ROWS_PT)])

            @pl.when(s == NS - 1)
            def _():
                pltpu.sync_copy(acc.at[pl.ds(REM_BASE, ROWS_REM)],
                                out_hbm.at[p, c, pl.ds(REM_BASE, ROWS_REM)])

    return edge_kernel


_edge_scatter_128 = _make_edge_scatter(HALF, 2)
_edge_scatter_16 = _make_edge_scatter(NUM_CLASSES, 1)


_BLK = 1000
_GRID = N_NODES // _BLK

_DINV_SPEC = pl.BlockSpec((1, 1, _BLK), lambda i: (i, 0, 0))


def _dinv_block(dinv_ref):
    return dinv_ref[...].reshape(_BLK)


def _tc_z_body(x_ref, w1_ref, z_ref):
    z_ref[...] = jnp.dot(x_ref[...], w1_ref[...],
                         preferred_element_type=jnp.float32)


def _tc_z(x, w1):
    return pl.pallas_call(
        _tc_z_body,
        grid=(_GRID,),
        in_specs=[
            pl.BlockSpec((_BLK, IN_FEAT), lambda i: (i, 0)),
            pl.BlockSpec((IN_FEAT, HIDDEN), lambda i: (0, 0)),
        ],
        out_specs=pl.BlockSpec((_BLK, HIDDEN), lambda i: (i, 0)),
        out_shape=jax.ShapeDtypeStruct((N_NODES, HIDDEN), jnp.float32),
    )(x, w1)


def _tc_y1_body(degp_ref, z_ref, y_ref, dinv_ref):
    deg = jnp.sum(degp_ref[0], axis=0) + 1.0
    dinv = lax.rsqrt(deg)
    y = z_ref[...] * dinv[:, None]
    y_ref[0] = y[:, :HALF]
    y_ref[1] = y[:, HALF:]
    dinv_ref[...] = dinv[None, None, :]


def _tc_y1(degp, z):
    return pl.pallas_call(
        _tc_y1_body,
        grid=(_GRID,),
        in_specs=[
            pl.BlockSpec((1, NW, _BLK), lambda i: (i, 0, 0)),
            pl.BlockSpec((_BLK, HIDDEN), lambda i: (i, 0)),
        ],
        out_specs=[
            pl.BlockSpec((2, _BLK, HALF), lambda i: (0, i, 0)),
            pl.BlockSpec((1, 1, _BLK), lambda i: (i, 0, 0)),
        ],
        out_shape=[
            jax.ShapeDtypeStruct((2, N_NODES, HALF), jnp.float32),
            jax.ShapeDtypeStruct((_GRID, 1, _BLK), jnp.float32),
        ],
    )(degp, z)


def _tc_mid_body(acc_ref, y1_ref, dinv_ref, w2_ref, w3_ref, y2_ref):
    dinv = _dinv_block(dinv_ref)
    h = jnp.concatenate(
        [acc_ref[0, 0] + acc_ref[0, 1] + y1_ref[0],
         acc_ref[1, 0] + acc_ref[1, 1] + y1_ref[1]], axis=1) * dinv[:, None]
    h = jnp.maximum(h, 0.0)
    w23 = jnp.dot(w2_ref[...], w3_ref[...], preferred_element_type=jnp.float32)
    y2_ref[0] = jnp.dot(h, w23, preferred_element_type=jnp.float32) * dinv[:, None]


def _tc_mid(acc, y1, dinv, w2, w3):
    return pl.pallas_call(
        _tc_mid_body,
        grid=(_GRID,),
        in_specs=[
            pl.BlockSpec((2, NC, _BLK, HALF), lambda i: (0, 0, i, 0)),
            pl.BlockSpec((2, _BLK, HALF), lambda i: (0, i, 0)),
            _DINV_SPEC,
            pl.BlockSpec((HIDDEN, HIDDEN), lambda i: (0, 0)),
            pl.BlockSpec((HIDDEN, NUM_CLASSES), lambda i: (0, 0)),
        ],
        out_specs=pl.BlockSpec((1, _BLK, NUM_CLASSES), lambda i: (0, i, 0)),
        out_shape=jax.ShapeDtypeStruct((1, N_NODES, NUM_CLASSES), jnp.float32),
    )(acc, y1, dinv, w2, w3)


def _tc_out_body(acc_ref, y2_ref, dinv_ref, o_ref):
    dinv = _dinv_block(dinv_ref)
    o_ref[...] = (acc_ref[0, 0] + acc_ref[0, 1] + y2_ref[0]) * dinv[:, None]


def _tc_out(acc, y2, dinv):
    return pl.pallas_call(
        _tc_out_body,
        grid=(_GRID,),
        in_specs=[
            pl.BlockSpec((1, NC, _BLK, NUM_CLASSES), lambda i: (0, 0, i, 0)),
            pl.BlockSpec((1, _BLK, NUM_CLASSES), lambda i: (0, i, 0)),
            _DINV_SPEC,
        ],
        out_specs=pl.BlockSpec((_BLK, NUM_CLASSES), lambda i: (i, 0)),
        out_shape=jax.ShapeDtypeStruct((N_NODES, NUM_CLASSES), jnp.float32),
    )(acc, y2, dinv)


def kernel(x, edge_index, W1, W2, W3):
    src = edge_index[0].astype(jnp.int32)
    dst = edge_index[1].astype(jnp.int32)
    src_p = jnp.concatenate(
        [src, jnp.zeros((E_PAD - N_EDGES,), jnp.int32)]).reshape(NW, CPT, K_CHUNK)
    dst_p = jnp.concatenate(
        [dst, jnp.full((E_PAD - N_EDGES,), N_NODES, jnp.int32)]).reshape(NW, CPT, K_CHUNK)
    zeros64 = jnp.zeros((ROWS_PT, HALF), jnp.float32)
    zeros16 = jnp.zeros((ROWS_PT, NUM_CLASSES), jnp.float32)

    degp = _deg_kernel(dst)
    z = _tc_z(x, W1)
    degp_t = degp.reshape(NW, _GRID, _BLK).transpose(1, 0, 2)
    y1, dinv = _tc_y1(degp_t, z)
    acc1 = _edge_scatter_128(y1, src_p, dst_p, zeros64)
    y2 = _tc_mid(acc1, y1, dinv, W2, W3)
    acc2 = _edge_scatter_16(y2, src_p, dst_p, zeros16)
    return _tc_out(acc2, y2, dinv)


# speedup vs baseline: 1.0141x; 1.0141x over previous
import functools

import jax
import jax.numpy as jnp
from jax import lax
from jax.experimental import pallas as pl
from jax.experimental.pallas import tpu as pltpu
from jax.experimental.pallas import tpu_sc as plsc

N_NODES = 10000
N_EDGES = 320000
IN_FEAT = 128
HIDDEN = 128
NUM_CLASSES = 16
HALF = HIDDEN // 2

NC = 2
NS = 16
NW = NC * NS
EPT = N_EDGES // NW
K_CHUNK = 128
CPT = 80
EPT_PAD = CPT * K_CHUNK
E_PAD = NW * EPT_PAD
N_ACC = N_NODES + 16
ROWS_PT = 624
ROWS_REM = N_NODES - NS * ROWS_PT
REM_BASE = NS * ROWS_PT

_MESH = plsc.VectorSubcoreMesh(core_axis_name="c", subcore_axis_name="s")


@functools.partial(
    pl.kernel,
    out_type=jax.ShapeDtypeStruct((10, NW, 1000), jnp.float32),
    mesh=_MESH,
    scratch_types=[
        pltpu.VMEM((EPT,), jnp.int32),
        pltpu.VMEM((N_NODES,), jnp.float32),
    ],
    compiler_params=pltpu.CompilerParams(needs_layout_passes=False,
                                         use_tc_tiling_on_sc=False),
)
def _deg_kernel(dst_hbm, out_hbm, idx_v, deg_v):
    c = lax.axis_index("c")
    s = lax.axis_index("s")
    wid = c * NS + s
    base = wid * EPT
    pltpu.sync_copy(dst_hbm.at[pl.ds(base, EPT)], idx_v)

    zeros16 = jnp.zeros((16,), jnp.float32)

    def zero_body(i, carry):
        deg_v[pl.ds(i * 16, 16)] = zeros16
        return carry

    lax.fori_loop(0, N_NODES // 16, zero_body, 0)

    ones16 = jnp.ones((16,), jnp.float32)

    def add_body(i, carry):
        idx = idx_v[pl.ds(i * 16, 16)]
        plsc.addupdate_scatter(deg_v, [idx], ones16)
        return carry

    lax.fori_loop(0, EPT // 16, add_body, 0)
    for i in range(10):
        pltpu.sync_copy(deg_v.at[pl.ds(i * 1000, 1000)], out_hbm.at[i, wid])


def _make_edge_scatter(feat, npass):
    @functools.partial(
        pl.kernel,
        out_type=jax.ShapeDtypeStruct((npass, NC, N_NODES, feat), jnp.float32),
        mesh=_MESH,
        compiler_params=pltpu.CompilerParams(use_tc_tiling_on_sc=False),
        scratch_types=[
            pltpu.VMEM((CPT, K_CHUNK), jnp.int32),
            pltpu.VMEM((CPT, K_CHUNK), jnp.int32),
            pltpu.VMEM((2, K_CHUNK, feat), jnp.float32),
            pltpu.VMEM_SHARED((N_NODES, feat), jnp.float32),
            pltpu.VMEM_SHARED((N_ACC, feat), jnp.float32),
            pltpu.SemaphoreType.DMA,
            pltpu.SemaphoreType.DMA,
        ],
    )
    def edge_kernel(y_hbm, src_hbm, dst_hbm, zeros_hbm, out_hbm,
                    sidx, didx, rows, y_s, acc, sem0, sem1):
        c = lax.axis_index("c")
        s = lax.axis_index("s")
        wid = c * NS + s
        rbase = s * ROWS_PT

        pltpu.sync_copy(src_hbm.at[wid], sidx)
        pltpu.sync_copy(dst_hbm.at[wid], didx)

        def fire(g, buf, sem):
            pltpu.async_copy(y_s.at[sidx.at[g]], rows.at[buf], sem)

        def wait(g, buf, sem):
            pltpu.make_async_copy(y_s.at[sidx.at[g]], rows.at[buf], sem).wait()

        def scatter(g, buf):
            pltpu.sync_copy(rows.at[buf], acc.at[didx.at[g]], add=True)

        for p in range(npass):
            pltpu.sync_copy(y_hbm.at[p, pl.ds(rbase, ROWS_PT)],
                            y_s.at[pl.ds(rbase, ROWS_PT)])

            @pl.when(c == 0)
            def _():
                pltpu.sync_copy(y_hbm.at[p, pl.ds(rbase, ROWS_PT)],
                                acc.at[pl.ds(rbase, ROWS_PT)])

            @pl.when(c != 0)
            def _():
                pltpu.sync_copy(zeros_hbm, acc.at[pl.ds(rbase, ROWS_PT)])

            @pl.when(s == NS - 1)
            def _():
                pltpu.sync_copy(y_hbm.at[p, pl.ds(REM_BASE, ROWS_REM)],
                                y_s.at[pl.ds(REM_BASE, ROWS_REM)])

                @pl.when(c == 0)
                def _():
                    pltpu.sync_copy(y_hbm.at[p, pl.ds(REM_BASE, ROWS_REM)],
                                    acc.at[pl.ds(REM_BASE, ROWS_REM)])

                @pl.when(c != 0)
                def _():
                    pltpu.sync_copy(zeros_hbm.at[pl.ds(0, ROWS_REM)],
                                    acc.at[pl.ds(REM_BASE, ROWS_REM)])

            plsc.subcore_barrier()

            fire(0, 0, sem0)

            def body(q, carry):
                g0 = 2 * q
                g1 = g0 + 1
                fire(g1, 1, sem1)
                wait(g0, 0, sem0)
                scatter(g0, 0)

                @pl.when(g1 + 1 < CPT)
                def _():
                    fire(g1 + 1, 0, sem0)

                wait(g1, 1, sem1)
                scatter(g1, 1)
                return carry

            lax.fori_loop(0, CPT // 2, body, 0)
            plsc.subcore_barrier()
            pltpu.sync_copy(acc.at[pl.ds(rbase, ROWS_PT)],
                            out_hbm.at[p, c, pl.ds(rbase, ROWS_PT)])

            @pl.when(s == NS - 1)
            def _():
                pltpu.sync_copy(acc.at[pl.ds(REM_BASE, ROWS_REM)],
                                out_hbm.at[p, c, pl.ds(REM_BASE, ROWS_REM)])

    return edge_kernel


_edge_scatter_128 = _make_edge_scatter(HALF, 2)
_edge_scatter_16 = _make_edge_scatter(NUM_CLASSES, 1)


_BLK = 1000
_GRID = N_NODES // _BLK

_DINV_SPEC = pl.BlockSpec((1, 1, _BLK), lambda i: (i, 0, 0))


def _dinv_block(dinv_ref):
    return dinv_ref[...].reshape(_BLK)


def _tc_z_body(x_ref, w1_ref, z_ref):
    z_ref[...] = jnp.dot(x_ref[...], w1_ref[...],
                         preferred_element_type=jnp.float32)


def _tc_z(x, w1):
    return pl.pallas_call(
        _tc_z_body,
        grid=(_GRID,),
        in_specs=[
            pl.BlockSpec((_BLK, IN_FEAT), lambda i: (i, 0)),
            pl.BlockSpec((IN_FEAT, HIDDEN), lambda i: (0, 0)),
        ],
        out_specs=pl.BlockSpec((_BLK, HIDDEN), lambda i: (i, 0)),
        out_shape=jax.ShapeDtypeStruct((N_NODES, HIDDEN), jnp.float32),
    )(x, w1)


def _tc_y1_body(degp_ref, z_ref, y_ref, dinv_ref):
    deg = jnp.sum(degp_ref[0], axis=0) + 1.0
    dinv = lax.rsqrt(deg)
    y = z_ref[...] * dinv[:, None]
    y_ref[0] = y[:, :HALF]
    y_ref[1] = y[:, HALF:]
    dinv_ref[...] = dinv[None, None, :]


def _tc_y1(degp, z):
    return pl.pallas_call(
        _tc_y1_body,
        grid=(_GRID,),
        in_specs=[
            pl.BlockSpec((1, NW, _BLK), lambda i: (i, 0, 0)),
            pl.BlockSpec((_BLK, HIDDEN), lambda i: (i, 0)),
        ],
        out_specs=[
            pl.BlockSpec((2, _BLK, HALF), lambda i: (0, i, 0)),
            pl.BlockSpec((1, 1, _BLK), lambda i: (i, 0, 0)),
        ],
        out_shape=[
            jax.ShapeDtypeStruct((2, N_NODES, HALF), jnp.float32),
            jax.ShapeDtypeStruct((_GRID, 1, _BLK), jnp.float32),
        ],
    )(degp, z)


def _tc_mid_body(acc_ref, dinv_ref, w2_ref, w3_ref, y2_ref):
    dinv = _dinv_block(dinv_ref)
    h = jnp.concatenate(
        [acc_ref[0, 0] + acc_ref[0, 1],
         acc_ref[1, 0] + acc_ref[1, 1]], axis=1) * dinv[:, None]
    h = jnp.maximum(h, 0.0)
    w23 = jnp.dot(w2_ref[...], w3_ref[...], preferred_element_type=jnp.float32)
    y2_ref[0] = jnp.dot(h, w23, preferred_element_type=jnp.float32) * dinv[:, None]


def _tc_mid(acc, dinv, w2, w3):
    return pl.pallas_call(
        _tc_mid_body,
        grid=(_GRID,),
        in_specs=[
            pl.BlockSpec((2, NC, _BLK, HALF), lambda i: (0, 0, i, 0)),
            _DINV_SPEC,
            pl.BlockSpec((HIDDEN, HIDDEN), lambda i: (0, 0)),
            pl.BlockSpec((HIDDEN, NUM_CLASSES), lambda i: (0, 0)),
        ],
        out_specs=pl.BlockSpec((1, _BLK, NUM_CLASSES), lambda i: (0, i, 0)),
        out_shape=jax.ShapeDtypeStruct((1, N_NODES, NUM_CLASSES), jnp.float32),
    )(acc, dinv, w2, w3)


def _tc_out_body(acc_ref, dinv_ref, o_ref):
    dinv = _dinv_block(dinv_ref)
    o_ref[...] = (acc_ref[0, 0] + acc_ref[0, 1]) * dinv[:, None]


def _tc_out(acc, dinv):
    return pl.pallas_call(
        _tc_out_body,
        grid=(_GRID,),
        in_specs=[
            pl.BlockSpec((1, NC, _BLK, NUM_CLASSES), lambda i: (0, 0, i, 0)),
            _DINV_SPEC,
        ],
        out_specs=pl.BlockSpec((_BLK, NUM_CLASSES), lambda i: (i, 0)),
        out_shape=jax.ShapeDtypeStruct((N_NODES, NUM_CLASSES), jnp.float32),
    )(acc, dinv)


def kernel(x, edge_index, W1, W2, W3):
    src = edge_index[0].astype(jnp.int32)
    dst = edge_index[1].astype(jnp.int32)
    src_p = jnp.concatenate(
        [src, jnp.zeros((E_PAD - N_EDGES,), jnp.int32)]).reshape(NW, CPT, K_CHUNK)
    dst_p = jnp.concatenate(
        [dst, jnp.full((E_PAD - N_EDGES,), N_NODES, jnp.int32)]).reshape(NW, CPT, K_CHUNK)
    zeros64 = jnp.zeros((ROWS_PT, HALF), jnp.float32)
    zeros16 = jnp.zeros((ROWS_PT, NUM_CLASSES), jnp.float32)

    degp = _deg_kernel(dst)
    z = _tc_z(x, W1)
    y1, dinv = _tc_y1(degp, z)
    acc1 = _edge_scatter_128(y1, src_p, dst_p, zeros64)
    y2 = _tc_mid(acc1, dinv, W2, W3)
    acc2 = _edge_scatter_16(y2, src_p, dst_p, zeros16)
    return _tc_out(acc2, dinv)


# speedup vs baseline: 1.0765x; 1.0615x over previous
import functools

import jax
import jax.numpy as jnp
from jax import lax
from jax.experimental import pallas as pl
from jax.experimental.pallas import tpu as pltpu
from jax.experimental.pallas import tpu_sc as plsc

N_NODES = 10000
N_EDGES = 320000
IN_FEAT = 128
HIDDEN = 128
NUM_CLASSES = 16
HALF = HIDDEN // 2

NC = 2
NS = 16
NW = NC * NS
EPT = N_EDGES // NW
K_CHUNK = 128
CPT = 80
EPT_PAD = CPT * K_CHUNK
E_PAD = NW * EPT_PAD
CPH = 80
CPT_F = 2 * CPH
E_PAD_F = NS * CPT_F * K_CHUNK
N_ACC = N_NODES + 16
ROWS_PT = 624
ROWS_REM = N_NODES - NS * ROWS_PT
REM_BASE = NS * ROWS_PT

_MESH = plsc.VectorSubcoreMesh(core_axis_name="c", subcore_axis_name="s")


@functools.partial(
    pl.kernel,
    out_type=jax.ShapeDtypeStruct((10, NW, 1000), jnp.float32),
    mesh=_MESH,
    scratch_types=[
        pltpu.VMEM((EPT,), jnp.int32),
        pltpu.VMEM((N_NODES,), jnp.float32),
    ],
    compiler_params=pltpu.CompilerParams(needs_layout_passes=False,
                                         use_tc_tiling_on_sc=False),
)
def _deg_kernel(dst_hbm, out_hbm, idx_v, deg_v):
    c = lax.axis_index("c")
    s = lax.axis_index("s")
    wid = c * NS + s
    base = wid * EPT
    pltpu.sync_copy(dst_hbm.at[pl.ds(base, EPT)], idx_v)

    zeros16 = jnp.zeros((16,), jnp.float32)

    def zero_body(i, carry):
        deg_v[pl.ds(i * 16, 16)] = zeros16
        return carry

    lax.fori_loop(0, N_NODES // 16, zero_body, 0)

    ones16 = jnp.ones((16,), jnp.float32)

    def add_body(i, carry):
        idx = idx_v[pl.ds(i * 16, 16)]
        plsc.addupdate_scatter(deg_v, [idx], ones16)
        return carry

    lax.fori_loop(0, EPT // 16, add_body, 0)
    for i in range(10):
        pltpu.sync_copy(deg_v.at[pl.ds(i * 1000, 1000)], out_hbm.at[i, wid])


def _make_edge_scatter(feat, npass):
    @functools.partial(
        pl.kernel,
        out_type=jax.ShapeDtypeStruct((npass, NC, N_NODES, feat), jnp.float32),
        mesh=_MESH,
        compiler_params=pltpu.CompilerParams(use_tc_tiling_on_sc=False),
        scratch_types=[
            pltpu.VMEM((CPT, K_CHUNK), jnp.int32),
            pltpu.VMEM((CPT, K_CHUNK), jnp.int32),
            pltpu.VMEM((2, K_CHUNK, feat), jnp.float32),
            pltpu.VMEM_SHARED((N_NODES, feat), jnp.float32),
            pltpu.VMEM_SHARED((N_ACC, feat), jnp.float32),
            pltpu.SemaphoreType.DMA,
            pltpu.SemaphoreType.DMA,
        ],
    )
    def edge_kernel(y_hbm, src_hbm, dst_hbm, zeros_hbm, out_hbm,
                    sidx, didx, rows, y_s, acc, sem0, sem1):
        c = lax.axis_index("c")
        s = lax.axis_index("s")
        wid = c * NS + s
        rbase = s * ROWS_PT

        pltpu.sync_copy(src_hbm.at[wid], sidx)
        pltpu.sync_copy(dst_hbm.at[wid], didx)

        def fire(g, buf, sem):
            pltpu.async_copy(y_s.at[sidx.at[g]], rows.at[buf], sem)

        def wait(g, buf, sem):
            pltpu.make_async_copy(y_s.at[sidx.at[g]], rows.at[buf], sem).wait()

        def scatter(g, buf):
            pltpu.sync_copy(rows.at[buf], acc.at[didx.at[g]], add=True)

        for p in range(npass):
            pltpu.sync_copy(y_hbm.at[p, pl.ds(rbase, ROWS_PT)],
                            y_s.at[pl.ds(rbase, ROWS_PT)])

            @pl.when(c == 0)
            def _():
                pltpu.sync_copy(y_hbm.at[p, pl.ds(rbase, ROWS_PT)],
                                acc.at[pl.ds(rbase, ROWS_PT)])

            @pl.when(c != 0)
            def _():
                pltpu.sync_copy(zeros_hbm, acc.at[pl.ds(rbase, ROWS_PT)])

            @pl.when(s == NS - 1)
            def _():
                pltpu.sync_copy(y_hbm.at[p, pl.ds(REM_BASE, ROWS_REM)],
                                y_s.at[pl.ds(REM_BASE, ROWS_REM)])

                @pl.when(c == 0)
                def _():
                    pltpu.sync_copy(y_hbm.at[p, pl.ds(REM_BASE, ROWS_REM)],
                                    acc.at[pl.ds(REM_BASE, ROWS_REM)])

                @pl.when(c != 0)
                def _():
                    pltpu.sync_copy(zeros_hbm.at[pl.ds(0, ROWS_REM)],
                                    acc.at[pl.ds(REM_BASE, ROWS_REM)])

            plsc.subcore_barrier()

            fire(0, 0, sem0)

            def body(q, carry):
                g0 = 2 * q
                g1 = g0 + 1
                fire(g1, 1, sem1)
                wait(g0, 0, sem0)
                scatter(g0, 0)

                @pl.when(g1 + 1 < CPT)
                def _():
                    fire(g1 + 1, 0, sem0)

                wait(g1, 1, sem1)
                scatter(g1, 1)
                return carry

            lax.fori_loop(0, CPT // 2, body, 0)
            plsc.subcore_barrier()
            pltpu.sync_copy(acc.at[pl.ds(rbase, ROWS_PT)],
                            out_hbm.at[p, c, pl.ds(rbase, ROWS_PT)])

            @pl.when(s == NS - 1)
            def _():
                pltpu.sync_copy(acc.at[pl.ds(REM_BASE, ROWS_REM)],
                                out_hbm.at[p, c, pl.ds(REM_BASE, ROWS_REM)])

    return edge_kernel


_edge_scatter_16 = _make_edge_scatter(NUM_CLASSES, 1)


@functools.partial(
    pl.kernel,
    out_type=jax.ShapeDtypeStruct((NC, N_NODES, HALF), jnp.float32),
    mesh=_MESH,
    compiler_params=pltpu.CompilerParams(use_tc_tiling_on_sc=False),
    scratch_types=[
        pltpu.VMEM((CPH, K_CHUNK), jnp.int32),
        pltpu.VMEM((CPH, K_CHUNK), jnp.int32),
        pltpu.VMEM((2, K_CHUNK, HALF), jnp.float32),
        pltpu.VMEM_SHARED((N_NODES, HALF), jnp.float32),
        pltpu.VMEM_SHARED((N_ACC, HALF), jnp.float32),
        pltpu.SemaphoreType.DMA,
        pltpu.SemaphoreType.DMA,
    ],
)
def _edge_scatter_fsplit(y_hbm, src_hbm, dst_hbm, out_hbm,
                         sidx, didx, rows, y_s, acc, sem0, sem1):
    c = lax.axis_index("c")
    s = lax.axis_index("s")
    rbase = s * ROWS_PT

    pltpu.sync_copy(y_hbm.at[c, pl.ds(rbase, ROWS_PT)],
                    y_s.at[pl.ds(rbase, ROWS_PT)])
    pltpu.sync_copy(y_hbm.at[c, pl.ds(rbase, ROWS_PT)],
                    acc.at[pl.ds(rbase, ROWS_PT)])

    @pl.when(s == NS - 1)
    def _():
        pltpu.sync_copy(y_hbm.at[c, pl.ds(REM_BASE, ROWS_REM)],
                        y_s.at[pl.ds(REM_BASE, ROWS_REM)])
        pltpu.sync_copy(y_hbm.at[c, pl.ds(REM_BASE, ROWS_REM)],
                        acc.at[pl.ds(REM_BASE, ROWS_REM)])

    plsc.subcore_barrier()

    def fire(g, buf, sem):
        pltpu.async_copy(y_s.at[sidx.at[g]], rows.at[buf], sem)

    def wait(g, buf, sem):
        pltpu.make_async_copy(y_s.at[sidx.at[g]], rows.at[buf], sem).wait()

    def scatter(g, buf):
        pltpu.sync_copy(rows.at[buf], acc.at[didx.at[g]], add=True)

    def body(q, carry):
        g0 = 2 * q
        g1 = g0 + 1
        fire(g1, 1, sem1)
        wait(g0, 0, sem0)
        scatter(g0, 0)

        @pl.when(g1 + 1 < CPH)
        def _():
            fire(g1 + 1, 0, sem0)

        wait(g1, 1, sem1)
        scatter(g1, 1)
        return carry

    for h in range(2):
        pltpu.sync_copy(src_hbm.at[s, pl.ds(h * CPH, CPH)], sidx)
        pltpu.sync_copy(dst_hbm.at[s, pl.ds(h * CPH, CPH)], didx)
        fire(0, 0, sem0)
        lax.fori_loop(0, CPH // 2, body, 0)

    plsc.subcore_barrier()
    pltpu.sync_copy(acc.at[pl.ds(rbase, ROWS_PT)],
                    out_hbm.at[c, pl.ds(rbase, ROWS_PT)])

    @pl.when(s == NS - 1)
    def _():
        pltpu.sync_copy(acc.at[pl.ds(REM_BASE, ROWS_REM)],
                        out_hbm.at[c, pl.ds(REM_BASE, ROWS_REM)])


_BLK = 1000
_GRID = N_NODES // _BLK

_DINV_SPEC = pl.BlockSpec((1, 1, _BLK), lambda i: (i, 0, 0))


def _dinv_block(dinv_ref):
    return dinv_ref[...].reshape(_BLK)


def _tc_y1_body(degp_ref, x_ref, w1_ref, y_ref, dinv_ref):
    deg = jnp.sum(degp_ref[0], axis=0) + 1.0
    dinv = lax.rsqrt(deg)
    z = jnp.dot(x_ref[...], w1_ref[...], preferred_element_type=jnp.float32)
    y = z * dinv[:, None]
    y_ref[0] = y[:, :HALF]
    y_ref[1] = y[:, HALF:]
    dinv_ref[...] = dinv[None, None, :]


def _tc_y1(degp, x, w1):
    return pl.pallas_call(
        _tc_y1_body,
        grid=(_GRID,),
        in_specs=[
            pl.BlockSpec((1, NW, _BLK), lambda i: (i, 0, 0)),
            pl.BlockSpec((_BLK, IN_FEAT), lambda i: (i, 0)),
            pl.BlockSpec((IN_FEAT, HIDDEN), lambda i: (0, 0)),
        ],
        out_specs=[
            pl.BlockSpec((2, _BLK, HALF), lambda i: (0, i, 0)),
            pl.BlockSpec((1, 1, _BLK), lambda i: (i, 0, 0)),
        ],
        out_shape=[
            jax.ShapeDtypeStruct((2, N_NODES, HALF), jnp.float32),
            jax.ShapeDtypeStruct((_GRID, 1, _BLK), jnp.float32),
        ],
    )(degp, x, w1)


def _tc_mid_body(acc_ref, dinv_ref, w2_ref, w3_ref, y2_ref):
    dinv = _dinv_block(dinv_ref)
    h = jnp.concatenate([acc_ref[0], acc_ref[1]], axis=1) * dinv[:, None]
    h = jnp.maximum(h, 0.0)
    w23 = jnp.dot(w2_ref[...], w3_ref[...], preferred_element_type=jnp.float32)
    y2_ref[0] = jnp.dot(h, w23, preferred_element_type=jnp.float32) * dinv[:, None]


def _tc_mid(acc, dinv, w2, w3):
    return pl.pallas_call(
        _tc_mid_body,
        grid=(_GRID,),
        in_specs=[
            pl.BlockSpec((NC, _BLK, HALF), lambda i: (0, i, 0)),
            _DINV_SPEC,
            pl.BlockSpec((HIDDEN, HIDDEN), lambda i: (0, 0)),
            pl.BlockSpec((HIDDEN, NUM_CLASSES), lambda i: (0, 0)),
        ],
        out_specs=pl.BlockSpec((1, _BLK, NUM_CLASSES), lambda i: (0, i, 0)),
        out_shape=jax.ShapeDtypeStruct((1, N_NODES, NUM_CLASSES), jnp.float32),
    )(acc, dinv, w2, w3)


def _tc_out_body(acc_ref, dinv_ref, o_ref):
    dinv = _dinv_block(dinv_ref)
    o_ref[...] = (acc_ref[0, 0] + acc_ref[0, 1]) * dinv[:, None]


def _tc_out(acc, dinv):
    return pl.pallas_call(
        _tc_out_body,
        grid=(_GRID,),
        in_specs=[
            pl.BlockSpec((1, NC, _BLK, NUM_CLASSES), lambda i: (0, 0, i, 0)),
            _DINV_SPEC,
        ],
        out_specs=pl.BlockSpec((_BLK, NUM_CLASSES), lambda i: (i, 0)),
        out_shape=jax.ShapeDtypeStruct((N_NODES, NUM_CLASSES), jnp.float32),
    )(acc, dinv)


def kernel(x, edge_index, W1, W2, W3):
    src = edge_index[0].astype(jnp.int32)
    dst = edge_index[1].astype(jnp.int32)
    src_p = jnp.concatenate(
        [src, jnp.zeros((E_PAD - N_EDGES,), jnp.int32)]).reshape(NW, CPT, K_CHUNK)
    dst_p = jnp.concatenate(
        [dst, jnp.full((E_PAD - N_EDGES,), N_NODES, jnp.int32)]).reshape(NW, CPT, K_CHUNK)
    src_f = jnp.concatenate(
        [src, jnp.zeros((E_PAD_F - N_EDGES,), jnp.int32)]).reshape(NS, CPT_F, K_CHUNK)
    dst_f = jnp.concatenate(
        [dst, jnp.full((E_PAD_F - N_EDGES,), N_NODES, jnp.int32)]).reshape(NS, CPT_F, K_CHUNK)
    zeros16 = jnp.zeros((ROWS_PT, NUM_CLASSES), jnp.float32)

    degp = _deg_kernel(dst)
    y1, dinv = _tc_y1(degp, x, W1)
    acc1 = _edge_scatter_fsplit(y1, src_f, dst_f)
    y2 = _tc_mid(acc1, dinv, W2, W3)
    acc2 = _edge_scatter_16(y2, src_p, dst_p, zeros16)
    return _tc_out(acc2, dinv)
